# baseline (speedup 1.0000x reference)
import functools

import jax
import jax.numpy as jnp
import numpy as np
from jax.experimental import pallas as pl
from jax.experimental.pallas import tpu as pltpu

_LN2 = 0.6931471805599453


def _comb(a1, b1, c1, d1, k1, a2, b2, c2, d2, k2):
    na = a2 * a1 + b2 * c1
    nb = a2 * b1 + b2 * d1
    nc = c2 * a1 + d2 * c1
    nd = c2 * b1 + d2 * d1
    m = jnp.maximum(jnp.maximum(na, nb), jnp.maximum(nc, nd))
    ebits = jax.lax.shift_right_logical(
        jax.lax.bitcast_convert_type(m, jnp.int32), 23)
    scale = jax.lax.bitcast_convert_type(
        jax.lax.shift_left(254 - ebits, 23), jnp.float32)
    nk = k1 + k2 + (ebits - 127)
    return na * scale, nb * scale, nc * scale, nd * scale, nk


def _body(lp_ref, lab_ref, lenr_ref, lenc_ref, w_ref, e_ref, sm_ref,
          out_ref, acc_ref, kacc_ref, *, n_b, t_blk, n_seg, nv):
    i = pl.program_id(0)
    nsteps = pl.num_programs(0)
    seglen = t_blk // n_seg

    e = lp_ref[...]
    p = jnp.exp(e).astype(jnp.bfloat16)

    s_all = jax.lax.dot_general(p, w_ref[...],
                                (((1,), (0,)), ((), ())),
                                preferred_element_type=jnp.float32)
    s = s_all[:, :3 * t_blk] + s_all[:, 3 * t_blk:]
    s00t = jnp.transpose(s[:, :t_blk], (1, 0))
    s10t = jnp.transpose(s[:, t_blk:2 * t_blk], (1, 0))
    p_it = jnp.transpose(s[:, 2 * t_blk:], (1, 0))

    labf = lab_ref[...].astype(jnp.bfloat16)
    labexp = jax.lax.dot_general(labf, e_ref[...],
                                 (((1,), (0,)), ((), ())),
                                 preferred_element_type=jnp.float32)
    c_iota = jax.lax.broadcasted_iota(jnp.int32, (n_b, t_blk * nv), 1)
    v_of_c = (c_iota % nv).astype(jnp.float32)
    t_of_c = c_iota // nv
    maskc = (i * t_blk + t_of_c) < lenc_ref[...]
    contrib = jnp.where((labexp == v_of_c) & maskc, e, 0.0)
    num_part = jnp.sum(contrib, axis=(0, 1), keepdims=True)

    r_iota = jax.lax.broadcasted_iota(jnp.int32, (t_blk, n_b), 0)
    mask = (i * t_blk + r_iota) < lenr_ref[...]
    a = jnp.where(mask, s00t, 1.0)
    b = jnp.where(mask, s10t, 0.0)
    c = jnp.where(mask, sm_ref[0] * p_it, 0.0)
    d = jnp.where(mask, sm_ref[1] * p_it, 1.0)

    a3 = jnp.transpose(a.reshape(n_seg, seglen, n_b), (1, 0, 2))
    b3 = jnp.transpose(b.reshape(n_seg, seglen, n_b), (1, 0, 2))
    c3 = jnp.transpose(c.reshape(n_seg, seglen, n_b), (1, 0, 2))
    d3 = jnp.transpose(d.reshape(n_seg, seglen, n_b), (1, 0, 2))

    ra, rb, rc, rd = a3[0], b3[0], c3[0], d3[0]
    rk = jnp.zeros((n_seg, n_b), jnp.int32)
    zk = rk
    for o in range(1, seglen):
        ra, rb, rc, rd, rk = _comb(ra, rb, rc, rd, rk,
                                   a3[o], b3[o], c3[o], d3[o], zk)

    pa, pb, pc, pd = ra[0:1], rb[0:1], rc[0:1], rd[0:1]
    pk = rk[0:1]
    for s2 in range(1, n_seg):
        pa, pb, pc, pd, pk = _comb(pa, pb, pc, pd, pk,
                                   ra[s2:s2 + 1], rb[s2:s2 + 1],
                                   rc[s2:s2 + 1], rd[s2:s2 + 1],
                                   rk[s2:s2 + 1])

    @pl.when(i == 0)
    def _init():
        acc_ref[0:1] = pa
        acc_ref[1:2] = pb
        acc_ref[2:3] = pc
        acc_ref[3:4] = pd
        acc_ref[4:5] = jnp.broadcast_to(num_part, (1, n_b))
        kacc_ref[0:1] = pk

    @pl.when(i > 0)
    def _accum():
        na, nb, nc, nd, nk = _comb(
            acc_ref[0:1], acc_ref[1:2], acc_ref[2:3], acc_ref[3:4],
            kacc_ref[0:1], pa, pb, pc, pd, pk)
        acc_ref[0:1] = na
        acc_ref[1:2] = nb
        acc_ref[2:3] = nc
        acc_ref[3:4] = nd
        acc_ref[4:5] = acc_ref[4:5] + jnp.broadcast_to(num_part, (1, n_b))
        kacc_ref[0:1] = nk

    @pl.when(i == nsteps - 1)
    def _final():
        den = (jnp.log(acc_ref[0:1]) + _LN2 * kacc_ref[0:1].astype(jnp.float32)
               + sm_ref[2])
        num_tot = acc_ref[4:5, 0:1]
        out_ref[...] = num_tot - jnp.sum(den, axis=(0, 1), keepdims=True)


def kernel(log_probs, input_lens, labels, den_scores):
    n_b, t_len, nv = log_probs.shape
    n_cls = (den_scores.shape[0] - 4) // 2
    t_blk = 128
    n_seg = 8
    n_chunk = t_len // t_blk

    w0 = jax.nn.log_softmax(den_scores[:n_cls + 3])
    w1 = jax.nn.log_softmax(den_scores[n_cls + 3:])
    sm = jnp.stack([jnp.exp(w0[1 + n_cls]),
                    jnp.exp(w1[n_cls]),
                    w0[2 + n_cls],
                    jnp.float32(0.0)])

    q0 = jnp.zeros((nv,), jnp.float32)
    q0 = q0.at[1].set(jnp.exp(w0[0]))
    q0 = q0.at[3:3 + n_cls].set(jnp.exp(w0[1:1 + n_cls]))
    q1 = jnp.zeros((nv,), jnp.float32)
    q1 = q1.at[3:3 + n_cls].set(jnp.exp(w1[:n_cls]))
    pi = jnp.zeros((nv,), jnp.float32).at[2].set(1.0)
    base = jnp.stack([q0, q1, pi], axis=1)
    eye = jnp.eye(t_blk, dtype=jnp.float32)
    w_f32 = (eye[:, None, None, :] * base[None, :, :, None]
             ).reshape(t_blk * nv, 3 * t_blk)
    w_hi = w_f32.astype(jnp.bfloat16)
    w_lo = (w_f32 - w_hi.astype(jnp.float32)).astype(jnp.bfloat16)
    w_mat = jnp.concatenate([w_hi, w_lo], axis=1)

    e_np = np.zeros((t_blk, t_blk * nv), np.float32)
    e_np[np.arange(t_blk * nv) // nv, np.arange(t_blk * nv)] = 1.0
    e_mat = jnp.asarray(e_np, dtype=jnp.bfloat16)

    lp2d = log_probs.reshape(n_b, t_len * nv)
    labs2d = labels.astype(jnp.int32)
    lens_r = input_lens.reshape(1, n_b).astype(jnp.int32)
    lens_c = input_lens.reshape(n_b, 1).astype(jnp.int32)

    res = pl.pallas_call(
        functools.partial(_body, n_b=n_b, t_blk=t_blk, n_seg=n_seg, nv=nv),
        grid=(n_chunk,),
        in_specs=[
            pl.BlockSpec((n_b, t_blk * nv), lambda i: (0, i)),
            pl.BlockSpec((n_b, t_blk), lambda i: (0, i)),
            pl.BlockSpec((1, n_b), lambda i: (0, 0)),
            pl.BlockSpec((n_b, 1), lambda i: (0, 0)),
            pl.BlockSpec((t_blk * nv, 6 * t_blk), lambda i: (0, 0)),
            pl.BlockSpec((t_blk, t_blk * nv), lambda i: (0, 0)),
            pl.BlockSpec(memory_space=pltpu.SMEM),
        ],
        out_specs=pl.BlockSpec((1, 1), lambda i: (0, 0)),
        out_shape=jax.ShapeDtypeStruct((1, 1), jnp.float32),
        scratch_shapes=[
            pltpu.VMEM((8, n_b), jnp.float32),
            pltpu.VMEM((8, n_b), jnp.int32),
        ],
    )(lp2d, labs2d, lens_r, lens_c, w_mat, e_mat, sm)
    return res[0, 0]


# speedup vs baseline: 1.6383x; 1.6383x over previous
import functools

import jax
import jax.numpy as jnp
from jax.experimental import pallas as pl
from jax.experimental.pallas import tpu as pltpu

_LN2 = 0.6931471805599453


def _comb(a1, b1, c1, d1, k1, a2, b2, c2, d2, k2):
    na = a2 * a1 + b2 * c1
    nb = a2 * b1 + b2 * d1
    nc = c2 * a1 + d2 * c1
    nd = c2 * b1 + d2 * d1
    m = jnp.maximum(jnp.maximum(na, nb), jnp.maximum(nc, nd))
    ebits = jax.lax.shift_right_logical(
        jax.lax.bitcast_convert_type(m, jnp.int32), 23)
    scale = jax.lax.bitcast_convert_type(
        jax.lax.shift_left(254 - ebits, 23), jnp.float32)
    nk = k1 + k2 + (ebits - 127)
    return na * scale, nb * scale, nc * scale, nd * scale, nk


def _body(lp_ref, lab_ref, len_ref, sm_ref, out_ref, acc_ref, kacc_ref,
          *, n_b, t_blk, n_seg):
    i = pl.program_id(0)
    nsteps = pl.num_programs(0)
    lens = len_ref[...]
    labs = lab_ref[...]
    nv = lp_ref.shape[1]
    seglen = t_blk // n_seg

    r_iota = jax.lax.broadcasted_iota(jnp.int32, (t_blk, n_b), 0)
    tglob = i * t_blk + r_iota
    mask = tglob < lens

    s00 = jnp.zeros((t_blk, n_b), jnp.float32)
    s10 = jnp.zeros((t_blk, n_b), jnp.float32)
    emit = jnp.zeros((t_blk, n_b), jnp.float32)
    p_i = None
    for v in range(nv):
        ev = lp_ref[0, v]
        emit = jnp.where(labs == v, ev, emit)
        if v == 0:
            continue
        pv = jnp.exp(ev)
        if v == 1:
            s00 = s00 + sm_ref[3] * pv
        elif v == 2:
            p_i = pv
        else:
            s00 = s00 + sm_ref[4 + v - 3] * pv
            s10 = s10 + sm_ref[14 + v - 3] * pv

    num_part = jnp.sum(jnp.where(mask, emit, 0.0), axis=0, keepdims=True)

    a = jnp.where(mask, s00, 1.0)
    b = jnp.where(mask, s10, 0.0)
    c = jnp.where(mask, sm_ref[0] * p_i, 0.0)
    d = jnp.where(mask, sm_ref[1] * p_i, 1.0)

    a3 = jnp.transpose(a.reshape(n_seg, seglen, n_b), (1, 0, 2))
    b3 = jnp.transpose(b.reshape(n_seg, seglen, n_b), (1, 0, 2))
    c3 = jnp.transpose(c.reshape(n_seg, seglen, n_b), (1, 0, 2))
    d3 = jnp.transpose(d.reshape(n_seg, seglen, n_b), (1, 0, 2))

    ra, rb, rc, rd = a3[0], b3[0], c3[0], d3[0]
    rk = jnp.zeros((n_seg, n_b), jnp.int32)
    zk = rk
    for o in range(1, seglen):
        ra, rb, rc, rd, rk = _comb(ra, rb, rc, rd, rk,
                                   a3[o], b3[o], c3[o], d3[o], zk)

    pa, pb, pc, pd = ra[0:1], rb[0:1], rc[0:1], rd[0:1]
    pk = rk[0:1]
    for s in range(1, n_seg):
        pa, pb, pc, pd, pk = _comb(pa, pb, pc, pd, pk,
                                   ra[s:s + 1], rb[s:s + 1],
                                   rc[s:s + 1], rd[s:s + 1], rk[s:s + 1])

    @pl.when(i == 0)
    def _init():
        acc_ref[0:1] = pa
        acc_ref[1:2] = pb
        acc_ref[2:3] = pc
        acc_ref[3:4] = pd
        acc_ref[4:5] = num_part
        kacc_ref[0:1] = pk

    @pl.when(i > 0)
    def _accum():
        na, nb, nc, nd, nk = _comb(
            acc_ref[0:1], acc_ref[1:2], acc_ref[2:3], acc_ref[3:4],
            kacc_ref[0:1], pa, pb, pc, pd, pk)
        acc_ref[0:1] = na
        acc_ref[1:2] = nb
        acc_ref[2:3] = nc
        acc_ref[3:4] = nd
        acc_ref[4:5] = acc_ref[4:5] + num_part
        kacc_ref[0:1] = nk

    @pl.when(i == nsteps - 1)
    def _final():
        den = (jnp.log(acc_ref[0:1]) + _LN2 * kacc_ref[0:1].astype(jnp.float32)
               + sm_ref[2])
        out_ref[...] = jnp.sum(acc_ref[4:5] - den, axis=(0, 1), keepdims=True)


def kernel(log_probs, input_lens, labels, den_scores):
    n_b, t_len, nv = log_probs.shape
    n_cls = (den_scores.shape[0] - 4) // 2
    t_blk = 512
    n_seg = 8
    seglen = t_blk // n_seg
    n_chunk = t_len // t_blk

    w0 = jax.nn.log_softmax(den_scores[:n_cls + 3])
    w1 = jax.nn.log_softmax(den_scores[n_cls + 3:])
    sm = jnp.concatenate([
        jnp.stack([jnp.exp(w0[1 + n_cls]),
                   jnp.exp(w1[n_cls]),
                   w0[2 + n_cls],
                   jnp.exp(w0[0])]),
        jnp.exp(w0[1:1 + n_cls]),
        jnp.exp(w1[:n_cls]),
    ])

    lp_t = (log_probs.reshape(n_b, n_chunk, t_blk, nv)
            .transpose(1, 3, 2, 0))
    labels_t = jnp.transpose(labels.astype(jnp.int32), (1, 0))
    lens2d = input_lens.reshape(1, n_b).astype(jnp.int32)

    res = pl.pallas_call(
        functools.partial(_body, n_b=n_b, t_blk=t_blk, n_seg=n_seg),
        grid=(n_chunk,),
        in_specs=[
            pl.BlockSpec((1, nv, t_blk, n_b), lambda i: (i, 0, 0, 0)),
            pl.BlockSpec((t_blk, n_b), lambda i: (i, 0)),
            pl.BlockSpec((1, n_b), lambda i: (0, 0)),
            pl.BlockSpec(memory_space=pltpu.SMEM),
        ],
        out_specs=pl.BlockSpec((1, 1), lambda i: (0, 0)),
        out_shape=jax.ShapeDtypeStruct((1, 1), jnp.float32),
        scratch_shapes=[
            pltpu.VMEM((8, n_b), jnp.float32),
            pltpu.VMEM((8, n_b), jnp.int32),
        ],
    )(lp_t, labels_t, lens2d, sm)
    return res[0, 0]


# speedup vs baseline: 2.2748x; 1.3885x over previous
import functools

import jax
import jax.numpy as jnp
from jax.experimental import pallas as pl
from jax.experimental.pallas import tpu as pltpu

_LN2 = 0.6931471805599453


def _comb(a1, b1, c1, d1, k1, a2, b2, c2, d2, k2):
    na = a2 * a1 + b2 * c1
    nb = a2 * b1 + b2 * d1
    nc = c2 * a1 + d2 * c1
    nd = c2 * b1 + d2 * d1
    m = jnp.maximum(jnp.maximum(na, nb), jnp.maximum(nc, nd))
    ebits = jax.lax.shift_right_logical(
        jax.lax.bitcast_convert_type(m, jnp.int32), 23)
    scale = jax.lax.bitcast_convert_type(
        jax.lax.shift_left(254 - ebits, 23), jnp.float32)
    nk = k1 + k2 + (ebits - 127)
    return na * scale, nb * scale, nc * scale, nd * scale, nk


def _body(lp_ref, lab_ref, len_ref, sm_ref, out_ref, acc_ref, kacc_ref,
          *, n_b, t_blk, n_seg):
    i = pl.program_id(0)
    nsteps = pl.num_programs(0)
    lens = len_ref[...]
    labs = lab_ref[...]
    nv = lp_ref.shape[0]
    seglen = t_blk // n_seg

    r_iota = jax.lax.broadcasted_iota(jnp.int32, (t_blk, n_b), 0)
    tglob = i * t_blk + r_iota
    mask = tglob < lens

    s00 = jnp.zeros((t_blk, n_b), jnp.float32)
    s10 = jnp.zeros((t_blk, n_b), jnp.float32)
    emit = jnp.zeros((t_blk, n_b), jnp.float32)
    p_i = None
    for v in range(nv):
        ev = lp_ref[v]
        emit = jnp.where(labs == v, ev, emit)
        if v == 0:
            continue
        pv = jnp.exp(ev)
        if v == 1:
            s00 = s00 + sm_ref[3] * pv
        elif v == 2:
            p_i = pv
        else:
            s00 = s00 + sm_ref[4 + v - 3] * pv
            s10 = s10 + sm_ref[14 + v - 3] * pv

    num_part = jnp.sum(jnp.where(mask, emit, 0.0), axis=0, keepdims=True)

    a = jnp.where(mask, s00, 1.0)
    b = jnp.where(mask, s10, 0.0)
    c = jnp.where(mask, sm_ref[0] * p_i, 0.0)
    d = jnp.where(mask, sm_ref[1] * p_i, 1.0)

    a3 = jnp.transpose(a.reshape(n_seg, seglen, n_b), (1, 0, 2))
    b3 = jnp.transpose(b.reshape(n_seg, seglen, n_b), (1, 0, 2))
    c3 = jnp.transpose(c.reshape(n_seg, seglen, n_b), (1, 0, 2))
    d3 = jnp.transpose(d.reshape(n_seg, seglen, n_b), (1, 0, 2))

    ra, rb, rc, rd = a3[0], b3[0], c3[0], d3[0]
    rk = jnp.zeros((n_seg, n_b), jnp.int32)
    zk = rk
    for o in range(1, seglen):
        ra, rb, rc, rd, rk = _comb(ra, rb, rc, rd, rk,
                                   a3[o], b3[o], c3[o], d3[o], zk)

    pa, pb, pc, pd = ra[0:1], rb[0:1], rc[0:1], rd[0:1]
    pk = rk[0:1]
    for s in range(1, n_seg):
        pa, pb, pc, pd, pk = _comb(pa, pb, pc, pd, pk,
                                   ra[s:s + 1], rb[s:s + 1],
                                   rc[s:s + 1], rd[s:s + 1], rk[s:s + 1])

    @pl.when(i == 0)
    def _init():
        acc_ref[0:1] = pa
        acc_ref[1:2] = pb
        acc_ref[2:3] = pc
        acc_ref[3:4] = pd
        acc_ref[4:5] = num_part
        kacc_ref[0:1] = pk

    @pl.when(i > 0)
    def _accum():
        na, nb, nc, nd, nk = _comb(
            acc_ref[0:1], acc_ref[1:2], acc_ref[2:3], acc_ref[3:4],
            kacc_ref[0:1], pa, pb, pc, pd, pk)
        acc_ref[0:1] = na
        acc_ref[1:2] = nb
        acc_ref[2:3] = nc
        acc_ref[3:4] = nd
        acc_ref[4:5] = acc_ref[4:5] + num_part
        kacc_ref[0:1] = nk

    @pl.when(i == nsteps - 1)
    def _final():
        den = (jnp.log(acc_ref[0:1]) + _LN2 * kacc_ref[0:1].astype(jnp.float32)
               + sm_ref[2])
        out_ref[...] = jnp.sum(acc_ref[4:5] - den, axis=(0, 1), keepdims=True)


def kernel(log_probs, input_lens, labels, den_scores):
    n_b, t_len, nv = log_probs.shape
    n_cls = (den_scores.shape[0] - 4) // 2
    t_blk = 512
    n_seg = 8
    seglen = t_blk // n_seg
    n_chunk = t_len // t_blk

    w0 = jax.nn.log_softmax(den_scores[:n_cls + 3])
    w1 = jax.nn.log_softmax(den_scores[n_cls + 3:])
    sm = jnp.concatenate([
        jnp.stack([jnp.exp(w0[1 + n_cls]),
                   jnp.exp(w1[n_cls]),
                   w0[2 + n_cls],
                   jnp.exp(w0[0])]),
        jnp.exp(w0[1:1 + n_cls]),
        jnp.exp(w1[:n_cls]),
    ])

    lp_t = jnp.transpose(log_probs, (2, 1, 0))
    labels_t = jnp.transpose(labels.astype(jnp.int32), (1, 0))
    lens2d = input_lens.reshape(1, n_b).astype(jnp.int32)

    res = pl.pallas_call(
        functools.partial(_body, n_b=n_b, t_blk=t_blk, n_seg=n_seg),
        grid=(n_chunk,),
        in_specs=[
            pl.BlockSpec((nv, t_blk, n_b), lambda i: (0, i, 0)),
            pl.BlockSpec((t_blk, n_b), lambda i: (i, 0)),
            pl.BlockSpec((1, n_b), lambda i: (0, 0)),
            pl.BlockSpec(memory_space=pltpu.SMEM),
        ],
        out_specs=pl.BlockSpec((1, 1), lambda i: (0, 0)),
        out_shape=jax.ShapeDtypeStruct((1, 1), jnp.float32),
        scratch_shapes=[
            pltpu.VMEM((8, n_b), jnp.float32),
            pltpu.VMEM((8, n_b), jnp.int32),
        ],
    )(lp_t, labels_t, lens2d, sm)
    return res[0, 0]
